# baseline (speedup 1.0000x reference)
import functools

import jax
import jax.numpy as jnp
import numpy as np
from jax import lax
from jax.experimental import pallas as pl
from jax.experimental.pallas import tpu as pltpu
from jax.experimental.pallas import tpu_sc as plsc

_NN = 10000
_W = 384
_NC = 2
_NWK = 32
_NRG = 64
_RPW = 160
_NR = _NRG * _RPW
_EB = 32
_PAD_SRC = 10008
_NP = 10240
_BLK = 256


def _scalar_from(vref, j):
    spl = plsc.load_gather(vref, [jnp.broadcast_to(j, (16,)).astype(jnp.int32)])
    return jnp.max(spl)


def _make_sc_seg(mode, width, wl_active, table_rows, size_e, out_dtype):
    wl = wl_active
    acc_len = (_RPW + 1) * width
    out_len = _NR * width
    mesh = plsc.VectorSubcoreMesh(core_axis_name="c", subcore_axis_name="s")
    gather = mode != "bit"

    scratch = [
        pltpu.VMEM((_EB,), jnp.int32),
        pltpu.VMEM((_EB,), jnp.int32),
        pltpu.VMEM((acc_len,), out_dtype),
        pltpu.VMEM((72,), jnp.int32),
        pltpu.VMEM((72,), jnp.int32),
    ]
    if gather:
        scratch += [pltpu.VMEM((_EB, width), out_dtype),
                    pltpu.SemaphoreType.DMA]

    def body(*refs):
        if gather:
            (table_h, srcp_h, ldst_h, bases_h, nbs_h, out_h,
             idx_v, ld_v, acc_v, bas_v, nbs_v, rows_v, sem) = refs
        else:
            (srcp_h, ldst_h, bases_h, nbs_h, out_h,
             idx_v, ld_v, acc_v, bas_v, nbs_v) = refs

        wid = lax.axis_index("s") * _NC + lax.axis_index("c")
        iota = lax.iota(jnp.int32, 16)
        zeros16 = jnp.zeros((16,), out_dtype)

        pltpu.sync_copy(bases_h, bas_v)
        pltpu.sync_copy(nbs_h, nbs_v)

        def range_body(rr, _c):
            rngid = wid * 2 + rr
            base = _scalar_from(bas_v, rngid)
            nb = _scalar_from(nbs_v, rngid)

            def zero_body(i, _):
                ix = jnp.broadcast_to(i * 16, (16,)).astype(jnp.int32) + iota
                plsc.store_scatter(acc_v, [ix], zeros16)
                return _

            lax.fori_loop(0, acc_len // 16, zero_body, 0)

            def edge_batch(b, _):
                off = pl.multiple_of(base + b * _EB, _EB)
                pltpu.sync_copy(srcp_h.at[pl.ds(off, _EB)], idx_v)
                pltpu.sync_copy(ldst_h.at[pl.ds(off, _EB)], ld_v)
                if gather:
                    pltpu.async_copy(table_h.at[idx_v], rows_v, sem).wait()

                def edge_one(e, _e):
                    fe = jnp.broadcast_to(e, (16,)).astype(jnp.int32)
                    ld = plsc.load_gather(ld_v, [fe])
                    rbase = ld * width
                    if mode == "bit":
                        s = plsc.load_gather(idx_v, [fe])
                        word = lax.shift_right_logical(s, 5)
                        bit = jnp.left_shift(jnp.ones_like(s), s & 31)
                        ix = rbase + word
                        cur = plsc.load_gather(acc_v, [ix])
                        plsc.store_scatter(acc_v, [ix], cur | bit)
                    else:
                        for w in range(wl):
                            ix = rbase + (w * 16) + iota
                            cur = plsc.load_gather(acc_v, [ix])
                            rw = plsc.load_gather(rows_v, [fe, (w * 16) + iota])
                            nv = (cur | rw) if mode == "or" else (cur + rw)
                            plsc.store_scatter(acc_v, [ix], nv)
                    return _e

                lax.fori_loop(0, _EB, edge_one, 0)
                return _

            lax.fori_loop(0, nb, edge_batch, 0)
            wb = pl.multiple_of(rngid * (_RPW * width), 8)
            pltpu.sync_copy(acc_v.at[pl.ds(0, _RPW * width)],
                            out_h.at[pl.ds(wb, _RPW * width)])
            return _c

        lax.fori_loop(0, 2, range_body, 0)

    kern = functools.partial(
        pl.kernel, mesh=mesh,
        out_type=jax.ShapeDtypeStruct((out_len,), out_dtype),
        scratch_types=scratch,
        compiler_params=pltpu.CompilerParams(needs_layout_passes=False))(body)
    return kern


def _ego_tc_body(m_ref, xre_ref, w_ref, b_ref, out_ref):
    m32 = m_ref[...]
    esum = jnp.zeros((_BLK, 128), jnp.float32)
    ecnt = jnp.zeros((_BLK, 1), jnp.float32)
    for b in range(32):
        bits = ((m32 >> b) & 1).astype(jnp.float32)
        esum += jnp.dot(bits, xre_ref[b],
                        preferred_element_type=jnp.float32)
        ecnt += jnp.sum(bits, axis=1, keepdims=True)
    ego = esum / jnp.maximum(ecnt, 1.0)
    out_ref[...] = jnp.dot(ego, w_ref[...],
                           preferred_element_type=jnp.float32) + b_ref[...]


def _ego_tc(m_pad, x_re, w_ego, b_ego2):
    return pl.pallas_call(
        _ego_tc_body,
        grid=(_NP // _BLK,),
        in_specs=[
            pl.BlockSpec((_BLK, _W), lambda i: (i, 0)),
            pl.BlockSpec((32, _W, 128), lambda i: (0, 0, 0)),
            pl.BlockSpec((128, 128), lambda i: (0, 0)),
            pl.BlockSpec((1, 128), lambda i: (0, 0)),
        ],
        out_specs=pl.BlockSpec((_BLK, 128), lambda i: (i, 0)),
        out_shape=jax.ShapeDtypeStruct((_NP, 128), jnp.float32),
    )(m_pad, x_re, w_ego, b_ego2)


def _cut_tc_body(c_ref, x_ref, w_ref, b_ref, out_ref):
    acc = c_ref[...]
    csum = acc[:, :128]
    ccnt = acc[:, 128:129]
    cut = jnp.where(ccnt > 0, csum / jnp.maximum(ccnt, 1.0), x_ref[...])
    out_ref[...] = jnp.dot(cut, w_ref[...],
                           preferred_element_type=jnp.float32) + b_ref[...]


def _cut_tc(cut_pad, x_pad, w_cut, b_cut2):
    return pl.pallas_call(
        _cut_tc_body,
        grid=(_NP // _BLK,),
        in_specs=[
            pl.BlockSpec((_BLK, 256), lambda i: (i, 0)),
            pl.BlockSpec((_BLK, 128), lambda i: (i, 0)),
            pl.BlockSpec((128, 128), lambda i: (0, 0)),
            pl.BlockSpec((1, 128), lambda i: (0, 0)),
        ],
        out_specs=pl.BlockSpec((_BLK, 128), lambda i: (i, 0)),
        out_shape=jax.ShapeDtypeStruct((_NP, 128), jnp.float32),
    )(cut_pad, x_pad, w_cut, b_cut2)


def _final_tc_body(agg_ref, x_ref, wg_ref, bg_ref, wf_ref, bf_ref, out_ref):
    agg = agg_ref[...]
    ego_enc = jnp.maximum(agg[:, :128], 0.0)
    cut_enc = jnp.maximum(agg[:, 128:], 0.0)
    glob = jnp.dot(x_ref[...], wg_ref[...],
                   preferred_element_type=jnp.float32) + bg_ref[...]
    comb = jnp.concatenate([ego_enc, cut_enc, glob], axis=1)
    o = jnp.dot(comb, wf_ref[...],
                preferred_element_type=jnp.float32) + bf_ref[...]
    mx = jnp.max(o, axis=1, keepdims=True)
    ls = jnp.log(jnp.sum(jnp.exp(o - mx), axis=1, keepdims=True))
    out_ref[...] = o - mx - ls


def _final_tc(agg_pad, x_pad, w_glob, b_glob2, w_fc, b_fc2):
    return pl.pallas_call(
        _final_tc_body,
        grid=(_NP // _BLK,),
        in_specs=[
            pl.BlockSpec((_BLK, 256), lambda i: (i, 0)),
            pl.BlockSpec((_BLK, 128), lambda i: (i, 0)),
            pl.BlockSpec((128, 128), lambda i: (0, 0)),
            pl.BlockSpec((1, 128), lambda i: (0, 0)),
            pl.BlockSpec((384, 128), lambda i: (0, 0)),
            pl.BlockSpec((1, 128), lambda i: (0, 0)),
        ],
        out_specs=pl.BlockSpec((_BLK, 128), lambda i: (i, 0)),
        out_shape=jax.ShapeDtypeStruct((_NP, 128), jnp.float32),
    )(agg_pad, x_pad, w_glob, b_glob2, w_fc, b_fc2)


def _partition(seg_sorted, other_sorted, self_mask=None):
    e = seg_sorted.shape[0]
    size = e + _NRG * _EB
    bounds = (jnp.arange(_NRG + 1, dtype=jnp.int32) * _RPW)
    starts = jnp.searchsorted(seg_sorted, bounds[:-1], side="left").astype(jnp.int32)
    ends = jnp.searchsorted(seg_sorted, bounds[1:], side="left").astype(jnp.int32)
    cnts = ends - starts
    pcnts = ((cnts + _EB - 1) // _EB) * _EB
    bases = jnp.concatenate([jnp.zeros((1,), jnp.int32),
                             jnp.cumsum(pcnts).astype(jnp.int32)])
    rng = seg_sorted // _RPW
    within = jnp.arange(e, dtype=jnp.int32) - starts[rng]
    pos = bases[rng] + within
    idx_p = jnp.full((size,), _PAD_SRC, jnp.int32).at[pos].set(other_sorted)
    ldst = seg_sorted - rng * _RPW
    if self_mask is not None:
        ldst = jnp.where(self_mask, _RPW, ldst)
    ldst_p = jnp.full((size,), _RPW, jnp.int32).at[pos].set(ldst)
    bases72 = jnp.zeros((72,), jnp.int32).at[:_NRG].set(bases[:_NRG])
    nbs72 = jnp.zeros((72,), jnp.int32).at[:_NRG].set(pcnts // _EB)
    return idx_p, ldst_p, bases72, nbs72


def kernel(x, edge_index, w_ego, b_ego, w_cut, b_cut, w_glob, b_glob, w_fc, b_fc):
    n = x.shape[0]
    src = edge_index[0].astype(jnp.int32)
    dst = edge_index[1].astype(jnp.int32)
    e = src.shape[0]

    ar = jnp.arange(n, dtype=jnp.int32)
    src_a = jnp.concatenate([src, ar])
    dst_a = jnp.concatenate([dst, ar])
    order = jnp.argsort(dst_a)
    dst_s = dst_a[order]
    src_s = src_a[order]
    self_m = order >= e
    srcP, ldstP, bases40, nbs40 = _partition(dst_s, src_s)
    _, ldstP_mp, _, _ = _partition(dst_s, src_s, self_mask=self_m)

    sc_bit = _make_sc_seg("bit", _W, 0, 0, srcP.shape[0], jnp.int32)
    b1_flat = sc_bit(srcP, ldstP, bases40, nbs40)
    b1 = b1_flat.reshape(_NR, _W)

    sc_or = _make_sc_seg("or", _W, 20, _NR, srcP.shape[0], jnp.int32)
    m_pad = sc_or(b1, srcP, ldstP, bases40, nbs40).reshape(_NR, _W)

    x_pad = jnp.zeros((_NP, 128), jnp.float32).at[:n].set(x)
    x_big = jnp.zeros((_W * 32, 128), jnp.float32).at[:n].set(x)
    x_re = x_big.reshape(_W, 32, 128).transpose(1, 0, 2)
    hl_ego = _ego_tc(m_pad, x_re, w_ego, b_ego.reshape(1, 128))

    perm = jax.random.permutation(jax.random.key(1), e)
    keep = perm[e // 2:]
    ks = src[keep]
    kd = dst[keep]
    korder = jnp.argsort(ks)
    ks_s = ks[korder]
    kd_s = kd[korder]
    kidxP, kldstP, kbases40, knbs40 = _partition(ks_s, kd_s)
    x_e = jnp.zeros((_NR, 256), jnp.float32).at[:n, :128].set(x)
    x_e = x_e.at[:n, 128].set(1.0)
    sc_add_cut = _make_sc_seg("add", 256, 9, _NR, kidxP.shape[0], jnp.float32)
    cut_pad = sc_add_cut(x_e, kidxP, kldstP, kbases40, knbs40).reshape(_NR, 256)

    hl_cut = _cut_tc(cut_pad, x_pad, w_cut, b_cut.reshape(1, 128))

    hl_cat = jnp.concatenate([hl_ego, hl_cut], axis=1)
    sc_add_mp = _make_sc_seg("add", 256, 16, _NR, srcP.shape[0], jnp.float32)
    agg_pad = sc_add_mp(hl_cat, srcP, ldstP_mp, bases40, nbs40).reshape(_NR, 256)

    out = _final_tc(agg_pad, x_pad, w_glob, b_glob.reshape(1, 128),
                    w_fc, b_fc.reshape(1, 128))
    return out[:n]


# speedup vs baseline: 2.3032x; 2.3032x over previous
import functools

import jax
import jax.numpy as jnp
import numpy as np
from jax import lax
from jax.experimental import pallas as pl
from jax.experimental.pallas import tpu as pltpu
from jax.experimental.pallas import tpu_sc as plsc

_NN = 10000
_W = 384
_NC = 2
_NWK = 32
_NRG = 64
_RPW = 160
_NR = _NRG * _RPW
_EB = 32
_PAD_SRC = 10008
_NP = 10240
_BLK = 256


def _scalar_from(vref, j):
    spl = plsc.load_gather(vref, [jnp.broadcast_to(j, (16,)).astype(jnp.int32)])
    return jnp.max(spl)


def _make_sc_seg(mode, width, wl_active, table_rows, size_e, out_dtype):
    wl = wl_active
    acc_len = (_RPW + 1) * width
    out_len = _NR * width
    mesh = plsc.VectorSubcoreMesh(core_axis_name="c", subcore_axis_name="s")
    gather = mode != "bit"

    scratch = [
        pltpu.VMEM((_EB,), jnp.int32),
        pltpu.VMEM((_EB,), jnp.int32),
        pltpu.VMEM((acc_len,), out_dtype),
        pltpu.VMEM((72,), jnp.int32),
        pltpu.VMEM((72,), jnp.int32),
    ]
    if gather:
        scratch += [pltpu.VMEM((_EB, width), out_dtype),
                    pltpu.SemaphoreType.DMA]

    def body(*refs):
        if gather:
            (table_h, srcp_h, ldst_h, bases_h, nbs_h, out_h,
             idx_v, ld_v, acc_v, bas_v, nbs_v, rows_v, sem) = refs
        else:
            (srcp_h, ldst_h, bases_h, nbs_h, out_h,
             idx_v, ld_v, acc_v, bas_v, nbs_v) = refs

        wid = lax.axis_index("s") * _NC + lax.axis_index("c")
        iota = lax.iota(jnp.int32, 16)
        zeros16 = jnp.zeros((16,), out_dtype)

        pltpu.sync_copy(bases_h, bas_v)
        pltpu.sync_copy(nbs_h, nbs_v)

        def range_body(rr, _c):
            rngid = wid * 2 + rr
            base = _scalar_from(bas_v, rngid)
            nb = _scalar_from(nbs_v, rngid)
            roff = jnp.broadcast_to(rngid * _RPW, (16,)).astype(jnp.int32)

            def zero_body(i, _):
                ix = jnp.broadcast_to(i * 16, (16,)).astype(jnp.int32) + iota
                plsc.store_scatter(acc_v, [ix], zeros16)
                return _

            lax.fori_loop(0, acc_len // 16, zero_body, 0)

            def edge_batch(b, _):
                off = pl.multiple_of(base + b * _EB, _EB)
                pltpu.sync_copy(srcp_h.at[pl.ds(off, _EB)], idx_v)
                pltpu.sync_copy(ldst_h.at[pl.ds(off, _EB)], ld_v)
                if gather:
                    pltpu.async_copy(table_h.at[idx_v], rows_v, sem).wait()

                def edge_one(e, _e):
                    fe = jnp.broadcast_to(e, (16,)).astype(jnp.int32)
                    dv = plsc.load_gather(ld_v, [fe])
                    ld = dv - roff
                    ok = (ld >= 0) & (ld < _RPW)
                    ld = jnp.where(ok, ld, _RPW)
                    rbase = ld * width
                    if mode == "bit":
                        s = plsc.load_gather(idx_v, [fe])
                        word = lax.shift_right_logical(s, 5)
                        bit = jnp.left_shift(jnp.ones_like(s), s & 31)
                        ix = rbase + word
                        cur = plsc.load_gather(acc_v, [ix])
                        plsc.store_scatter(acc_v, [ix], cur | bit)
                    else:
                        for w in range(wl):
                            ix = rbase + (w * 16) + iota
                            cur = plsc.load_gather(acc_v, [ix])
                            rw = plsc.load_gather(rows_v, [fe, (w * 16) + iota])
                            nv = (cur | rw) if mode == "or" else (cur + rw)
                            plsc.store_scatter(acc_v, [ix], nv)
                    return _e

                lax.fori_loop(0, _EB, edge_one, 0)
                return _

            lax.fori_loop(0, nb, edge_batch, 0)
            wb = pl.multiple_of(rngid * (_RPW * width), 8)
            pltpu.sync_copy(acc_v.at[pl.ds(0, _RPW * width)],
                            out_h.at[pl.ds(wb, _RPW * width)])
            return _c

        lax.fori_loop(0, 2, range_body, 0)

    kern = functools.partial(
        pl.kernel, mesh=mesh,
        out_type=jax.ShapeDtypeStruct((out_len,), out_dtype),
        scratch_types=scratch,
        compiler_params=pltpu.CompilerParams(needs_layout_passes=False))(body)
    return kern


def _ego_tc_body(m_ref, xre_ref, w_ref, b_ref, out_ref):
    m32 = m_ref[...]
    esum = jnp.zeros((_BLK, 128), jnp.float32)
    ecnt = jnp.zeros((_BLK, 1), jnp.float32)
    for b in range(32):
        bits = ((m32 >> b) & 1).astype(jnp.float32)
        esum += jnp.dot(bits, xre_ref[b],
                        preferred_element_type=jnp.float32)
        ecnt += jnp.sum(bits, axis=1, keepdims=True)
    ego = esum / jnp.maximum(ecnt, 1.0)
    out_ref[...] = jnp.dot(ego, w_ref[...],
                           preferred_element_type=jnp.float32) + b_ref[...]


def _ego_tc(m_pad, x_re, w_ego, b_ego2):
    return pl.pallas_call(
        _ego_tc_body,
        grid=(_NP // _BLK,),
        in_specs=[
            pl.BlockSpec((_BLK, _W), lambda i: (i, 0)),
            pl.BlockSpec((32, _W, 128), lambda i: (0, 0, 0)),
            pl.BlockSpec((128, 128), lambda i: (0, 0)),
            pl.BlockSpec((1, 128), lambda i: (0, 0)),
        ],
        out_specs=pl.BlockSpec((_BLK, 128), lambda i: (i, 0)),
        out_shape=jax.ShapeDtypeStruct((_NP, 128), jnp.float32),
    )(m_pad, x_re, w_ego, b_ego2)


def _cut_tc_body(c_ref, x_ref, w_ref, b_ref, out_ref):
    acc = c_ref[...]
    csum = acc[:, :128]
    ccnt = acc[:, 128:129]
    cut = jnp.where(ccnt > 0, csum / jnp.maximum(ccnt, 1.0), x_ref[...])
    out_ref[...] = jnp.dot(cut, w_ref[...],
                           preferred_element_type=jnp.float32) + b_ref[...]


def _cut_tc(cut_pad, x_pad, w_cut, b_cut2):
    return pl.pallas_call(
        _cut_tc_body,
        grid=(_NP // _BLK,),
        in_specs=[
            pl.BlockSpec((_BLK, 256), lambda i: (i, 0)),
            pl.BlockSpec((_BLK, 128), lambda i: (i, 0)),
            pl.BlockSpec((128, 128), lambda i: (0, 0)),
            pl.BlockSpec((1, 128), lambda i: (0, 0)),
        ],
        out_specs=pl.BlockSpec((_BLK, 128), lambda i: (i, 0)),
        out_shape=jax.ShapeDtypeStruct((_NP, 128), jnp.float32),
    )(cut_pad, x_pad, w_cut, b_cut2)


def _final_tc_body(agg_ref, x_ref, wg_ref, bg_ref, wf_ref, bf_ref, out_ref):
    agg = agg_ref[...]
    ego_enc = jnp.maximum(agg[:, :128], 0.0)
    cut_enc = jnp.maximum(agg[:, 128:], 0.0)
    glob = jnp.dot(x_ref[...], wg_ref[...],
                   preferred_element_type=jnp.float32) + bg_ref[...]
    comb = jnp.concatenate([ego_enc, cut_enc, glob], axis=1)
    o = jnp.dot(comb, wf_ref[...],
                preferred_element_type=jnp.float32) + bf_ref[...]
    mx = jnp.max(o, axis=1, keepdims=True)
    ls = jnp.log(jnp.sum(jnp.exp(o - mx), axis=1, keepdims=True))
    out_ref[...] = o - mx - ls


def _final_tc(agg_pad, x_pad, w_glob, b_glob2, w_fc, b_fc2):
    return pl.pallas_call(
        _final_tc_body,
        grid=(_NP // _BLK,),
        in_specs=[
            pl.BlockSpec((_BLK, 256), lambda i: (i, 0)),
            pl.BlockSpec((_BLK, 128), lambda i: (i, 0)),
            pl.BlockSpec((128, 128), lambda i: (0, 0)),
            pl.BlockSpec((1, 128), lambda i: (0, 0)),
            pl.BlockSpec((384, 128), lambda i: (0, 0)),
            pl.BlockSpec((1, 128), lambda i: (0, 0)),
        ],
        out_specs=pl.BlockSpec((_BLK, 128), lambda i: (i, 0)),
        out_shape=jax.ShapeDtypeStruct((_NP, 128), jnp.float32),
    )(agg_pad, x_pad, w_glob, b_glob2, w_fc, b_fc2)


def _spans(seg_sorted):
    bounds = (jnp.arange(_NRG + 1, dtype=jnp.int32) * _RPW)
    starts = jnp.searchsorted(seg_sorted, bounds, side="left").astype(jnp.int32)
    base = (starts[:-1] // 8) * 8
    nb = (starts[1:] - base + _EB - 1) // _EB
    bases72 = jnp.zeros((72,), jnp.int32).at[:_NRG].set(base)
    nbs72 = jnp.zeros((72,), jnp.int32).at[:_NRG].set(nb)
    return bases72, nbs72


def _pad_tail(a, fill):
    return jnp.concatenate([a, jnp.full((_NRG * _EB,), fill, jnp.int32)])


def kernel(x, edge_index, w_ego, b_ego, w_cut, b_cut, w_glob, b_glob, w_fc, b_fc):
    n = x.shape[0]
    src = edge_index[0].astype(jnp.int32)
    dst = edge_index[1].astype(jnp.int32)
    e = src.shape[0]

    ar = jnp.arange(n, dtype=jnp.int32)
    src_a = jnp.concatenate([src, ar])
    dst_a = jnp.concatenate([dst, ar])
    selff = jnp.concatenate([jnp.zeros((e,), jnp.int32), jnp.ones((n,), jnp.int32)])
    key = (dst_a << 15) | (selff << 14) | src_a
    key_s = jnp.sort(key)
    src_s = key_s & 16383
    selfb = (key_s >> 14) & 1
    dst_s = key_s >> 15
    dst_mp = jnp.where(selfb == 1, 20000, dst_s)
    srcP = _pad_tail(src_s, _PAD_SRC)
    dstP = _pad_tail(dst_s, 20000)
    dstmpP = _pad_tail(dst_mp, 20000)
    bases40, nbs40 = _spans(dst_s)

    sc_bit = _make_sc_seg("bit", _W, 0, 0, srcP.shape[0], jnp.int32)
    b1_flat = sc_bit(srcP, dstP, bases40, nbs40)
    b1 = b1_flat.reshape(_NR, _W)

    sc_or = _make_sc_seg("or", _W, 20, _NR, srcP.shape[0], jnp.int32)
    m_pad = sc_or(b1, srcP, dstP, bases40, nbs40).reshape(_NR, _W)

    x_pad = jnp.zeros((_NP, 128), jnp.float32).at[:n].set(x)
    x_big = jnp.zeros((_W * 32, 128), jnp.float32).at[:n].set(x)
    x_re = x_big.reshape(_W, 32, 128).transpose(1, 0, 2)
    hl_ego = _ego_tc(m_pad, x_re, w_ego, b_ego.reshape(1, 128))

    perm = jax.random.permutation(jax.random.key(1), e)
    keep = perm[e // 2:]
    ks = src[keep]
    kd = dst[keep]
    keyc = (ks << 14) | kd
    keyc_s = jnp.sort(keyc)
    kd_s = keyc_s & 16383
    ks_s = keyc_s >> 14
    kidxP = _pad_tail(kd_s, _PAD_SRC)
    kdstP = _pad_tail(ks_s, 20000)
    kbases40, knbs40 = _spans(ks_s)
    x_e = jnp.zeros((_NR, 256), jnp.float32).at[:n, :128].set(x)
    x_e = x_e.at[:n, 128].set(1.0)
    sc_add_cut = _make_sc_seg("add", 256, 9, _NR, kidxP.shape[0], jnp.float32)
    cut_pad = sc_add_cut(x_e, kidxP, kdstP, kbases40, knbs40).reshape(_NR, 256)

    hl_cut = _cut_tc(cut_pad, x_pad, w_cut, b_cut.reshape(1, 128))

    hl_cat = jnp.concatenate([hl_ego, hl_cut], axis=1)
    sc_add_mp = _make_sc_seg("add", 256, 16, _NR, srcP.shape[0], jnp.float32)
    agg_pad = sc_add_mp(hl_cat, srcP, dstmpP, bases40, nbs40).reshape(_NR, 256)

    out = _final_tc(agg_pad, x_pad, w_glob, b_glob.reshape(1, 128),
                    w_fc, b_fc.reshape(1, 128))
    return out[:n]


# speedup vs baseline: 2.5808x; 1.1205x over previous
import functools

import jax
import jax.numpy as jnp
import numpy as np
from jax import lax
from jax.experimental import pallas as pl
from jax.experimental.pallas import tpu as pltpu
from jax.experimental.pallas import tpu_sc as plsc

_NN = 10000
_W = 384
_NC = 2
_NWK = 32
_NRG = 64
_RPW = 160
_NR = _NRG * _RPW
_EB = 64
_PAD_SRC = 10008
_NP = 10240
_BLK = 256


def _scalar_from(vref, j):
    spl = plsc.load_gather(vref, [jnp.broadcast_to(j, (16,)).astype(jnp.int32)])
    return jnp.max(spl)


def _make_sc_seg(mode, width, wl_active, table_rows, size_e, out_dtype):
    wl = wl_active
    acc_len = (_RPW + 1) * width
    out_len = _NR * width
    mesh = plsc.VectorSubcoreMesh(core_axis_name="c", subcore_axis_name="s")
    gather = mode != "bit"

    scratch = [
        pltpu.VMEM((_EB,), jnp.int32),
        pltpu.VMEM((_EB,), jnp.int32),
        pltpu.VMEM((acc_len,), out_dtype),
        pltpu.VMEM((72,), jnp.int32),
        pltpu.VMEM((72,), jnp.int32),
    ]
    if gather:
        scratch += [pltpu.VMEM((_EB, width), out_dtype),
                    pltpu.SemaphoreType.DMA]

    def body(*refs):
        if gather:
            (table_h, srcp_h, ldst_h, bases_h, nbs_h, out_h,
             idx_v, ld_v, acc_v, bas_v, nbs_v, rows_v, sem) = refs
        else:
            (srcp_h, ldst_h, bases_h, nbs_h, out_h,
             idx_v, ld_v, acc_v, bas_v, nbs_v) = refs

        wid = lax.axis_index("s") * _NC + lax.axis_index("c")
        iota = lax.iota(jnp.int32, 16)
        zeros16 = jnp.zeros((16,), out_dtype)

        pltpu.sync_copy(bases_h, bas_v)
        pltpu.sync_copy(nbs_h, nbs_v)

        def range_body(rr, _c):
            rngid = wid * 2 + rr
            base = _scalar_from(bas_v, rngid)
            nb = _scalar_from(nbs_v, rngid)
            roff = jnp.broadcast_to(rngid * _RPW, (16,)).astype(jnp.int32)

            def zero_body(i, _):
                ix = jnp.broadcast_to(i * 16, (16,)).astype(jnp.int32) + iota
                plsc.store_scatter(acc_v, [ix], zeros16)
                return _

            lax.fori_loop(0, acc_len // 16, zero_body, 0)

            def edge_batch(b, _):
                off = pl.multiple_of(base + b * _EB, _EB)
                pltpu.sync_copy(srcp_h.at[pl.ds(off, _EB)], idx_v)
                pltpu.sync_copy(ldst_h.at[pl.ds(off, _EB)], ld_v)
                if gather:
                    pltpu.async_copy(table_h.at[idx_v], rows_v, sem).wait()

                def edge_one(e, _e):
                    fe = jnp.broadcast_to(e, (16,)).astype(jnp.int32)
                    dv = plsc.load_gather(ld_v, [fe])
                    ld = dv - roff
                    ok = (ld >= 0) & (ld < _RPW)
                    ld = jnp.where(ok, ld, _RPW)
                    rbase = ld * width
                    if mode == "bit":
                        s = plsc.load_gather(idx_v, [fe])
                        word = lax.shift_right_logical(s, 5)
                        bit = jnp.left_shift(jnp.ones_like(s), s & 31)
                        ix = rbase + word
                        cur = plsc.load_gather(acc_v, [ix])
                        plsc.store_scatter(acc_v, [ix], cur | bit)
                    else:
                        for w in range(wl):
                            ix = rbase + (w * 16) + iota
                            cur = plsc.load_gather(acc_v, [ix])
                            rw = plsc.load_gather(rows_v, [fe, (w * 16) + iota])
                            nv = (cur | rw) if mode == "or" else (cur + rw)
                            plsc.store_scatter(acc_v, [ix], nv)
                    return _e

                lax.fori_loop(0, _EB, edge_one, 0)
                return _

            lax.fori_loop(0, nb, edge_batch, 0)
            wb = pl.multiple_of(rngid * (_RPW * width), 8)
            pltpu.sync_copy(acc_v.at[pl.ds(0, _RPW * width)],
                            out_h.at[pl.ds(wb, _RPW * width)])
            return _c

        lax.fori_loop(0, 2, range_body, 0)

    kern = functools.partial(
        pl.kernel, mesh=mesh,
        out_type=jax.ShapeDtypeStruct((out_len,), out_dtype),
        scratch_types=scratch,
        compiler_params=pltpu.CompilerParams(needs_layout_passes=False))(body)
    return kern


def _ego_tc_body(m_ref, xre_ref, w_ref, b_ref, out_ref):
    m32 = m_ref[...][:, :320]
    esum = jnp.zeros((_BLK, 128), jnp.float32)
    ecnt = jnp.zeros((_BLK, 1), jnp.float32)
    for b in range(32):
        bits = ((m32 >> b) & 1).astype(jnp.float32)
        esum += jnp.dot(bits, xre_ref[b],
                        preferred_element_type=jnp.float32)
        ecnt += jnp.sum(bits, axis=1, keepdims=True)
    ego = esum / jnp.maximum(ecnt, 1.0)
    out_ref[...] = jnp.dot(ego, w_ref[...],
                           preferred_element_type=jnp.float32) + b_ref[...]


def _ego_tc(m_pad, x_re, w_ego, b_ego2):
    return pl.pallas_call(
        _ego_tc_body,
        grid=(_NP // _BLK,),
        in_specs=[
            pl.BlockSpec((_BLK, _W), lambda i: (i, 0)),
            pl.BlockSpec((32, 320, 128), lambda i: (0, 0, 0)),
            pl.BlockSpec((128, 128), lambda i: (0, 0)),
            pl.BlockSpec((1, 128), lambda i: (0, 0)),
        ],
        out_specs=pl.BlockSpec((_BLK, 128), lambda i: (i, 0)),
        out_shape=jax.ShapeDtypeStruct((_NP, 128), jnp.float32),
    )(m_pad, x_re, w_ego, b_ego2)


def _cut_tc_body(c_ref, x_ref, w_ref, b_ref, out_ref):
    acc = c_ref[...]
    csum = acc[:, :128]
    ccnt = acc[:, 128:129]
    cut = jnp.where(ccnt > 0, csum / jnp.maximum(ccnt, 1.0), x_ref[...])
    out_ref[...] = jnp.dot(cut, w_ref[...],
                           preferred_element_type=jnp.float32) + b_ref[...]


def _cut_tc(cut_pad, x_pad, w_cut, b_cut2):
    return pl.pallas_call(
        _cut_tc_body,
        grid=(_NP // _BLK,),
        in_specs=[
            pl.BlockSpec((_BLK, 256), lambda i: (i, 0)),
            pl.BlockSpec((_BLK, 128), lambda i: (i, 0)),
            pl.BlockSpec((128, 128), lambda i: (0, 0)),
            pl.BlockSpec((1, 128), lambda i: (0, 0)),
        ],
        out_specs=pl.BlockSpec((_BLK, 128), lambda i: (i, 0)),
        out_shape=jax.ShapeDtypeStruct((_NP, 128), jnp.float32),
    )(cut_pad, x_pad, w_cut, b_cut2)


def _final_tc_body(agg_ref, x_ref, wg_ref, bg_ref, wf_ref, bf_ref, out_ref):
    agg = agg_ref[...]
    ego_enc = jnp.maximum(agg[:, :128], 0.0)
    cut_enc = jnp.maximum(agg[:, 128:], 0.0)
    glob = jnp.dot(x_ref[...], wg_ref[...],
                   preferred_element_type=jnp.float32) + bg_ref[...]
    comb = jnp.concatenate([ego_enc, cut_enc, glob], axis=1)
    o = jnp.dot(comb, wf_ref[...],
                preferred_element_type=jnp.float32) + bf_ref[...]
    mx = jnp.max(o, axis=1, keepdims=True)
    ls = jnp.log(jnp.sum(jnp.exp(o - mx), axis=1, keepdims=True))
    out_ref[...] = o - mx - ls


def _final_tc(agg_pad, x_pad, w_glob, b_glob2, w_fc, b_fc2):
    return pl.pallas_call(
        _final_tc_body,
        grid=(_NP // _BLK,),
        in_specs=[
            pl.BlockSpec((_BLK, 256), lambda i: (i, 0)),
            pl.BlockSpec((_BLK, 128), lambda i: (i, 0)),
            pl.BlockSpec((128, 128), lambda i: (0, 0)),
            pl.BlockSpec((1, 128), lambda i: (0, 0)),
            pl.BlockSpec((384, 128), lambda i: (0, 0)),
            pl.BlockSpec((1, 128), lambda i: (0, 0)),
        ],
        out_specs=pl.BlockSpec((_BLK, 128), lambda i: (i, 0)),
        out_shape=jax.ShapeDtypeStruct((_NP, 128), jnp.float32),
    )(agg_pad, x_pad, w_glob, b_glob2, w_fc, b_fc2)


def _spans(seg_sorted):
    bounds = (jnp.arange(_NRG + 1, dtype=jnp.int32) * _RPW)
    starts = jnp.searchsorted(seg_sorted, bounds, side="left").astype(jnp.int32)
    base = (starts[:-1] // 8) * 8
    nb = (starts[1:] - base + _EB - 1) // _EB
    bases72 = jnp.zeros((72,), jnp.int32).at[:_NRG].set(base)
    nbs72 = jnp.zeros((72,), jnp.int32).at[:_NRG].set(nb)
    return bases72, nbs72


def _pad_tail(a, fill):
    return jnp.concatenate([a, jnp.full((_NRG * _EB,), fill, jnp.int32)])


def kernel(x, edge_index, w_ego, b_ego, w_cut, b_cut, w_glob, b_glob, w_fc, b_fc):
    n = x.shape[0]
    src = edge_index[0].astype(jnp.int32)
    dst = edge_index[1].astype(jnp.int32)
    e = src.shape[0]

    ar = jnp.arange(n, dtype=jnp.int32)
    src_a = jnp.concatenate([src, ar])
    dst_a = jnp.concatenate([dst, ar])
    selff = jnp.concatenate([jnp.zeros((e,), jnp.int32), jnp.ones((n,), jnp.int32)])
    key = (dst_a << 15) | (selff << 14) | src_a
    key_s = jnp.sort(key)
    src_s = key_s & 16383
    selfb = (key_s >> 14) & 1
    dst_s = key_s >> 15
    dst_mp = jnp.where(selfb == 1, 20000, dst_s)
    srcP = _pad_tail(src_s, _PAD_SRC)
    dstP = _pad_tail(dst_s, 20000)
    dstmpP = _pad_tail(dst_mp, 20000)
    bases40, nbs40 = _spans(dst_s)

    sc_bit = _make_sc_seg("bit", _W, 0, 0, srcP.shape[0], jnp.int32)
    b1_flat = sc_bit(srcP, dstP, bases40, nbs40)
    b1 = b1_flat.reshape(_NR, _W)

    sc_or = _make_sc_seg("or", _W, 20, _NR, srcP.shape[0], jnp.int32)
    m_pad = sc_or(b1, srcP, dstP, bases40, nbs40).reshape(_NR, _W)

    x_pad = jnp.zeros((_NP, 128), jnp.float32).at[:n].set(x)
    x_re = x_pad.reshape(320, 32, 128).transpose(1, 0, 2)
    hl_ego = _ego_tc(m_pad, x_re, w_ego, b_ego.reshape(1, 128))

    perm = jax.random.permutation(jax.random.key(1), e)
    keep = perm[e // 2:]
    ks = src[keep]
    kd = dst[keep]
    keyc = (ks << 14) | kd
    keyc_s = jnp.sort(keyc)
    kd_s = keyc_s & 16383
    ks_s = keyc_s >> 14
    kidxP = _pad_tail(kd_s, _PAD_SRC)
    kdstP = _pad_tail(ks_s, 20000)
    kbases40, knbs40 = _spans(ks_s)
    x_e = jnp.zeros((_NR, 256), jnp.float32).at[:n, :128].set(x)
    x_e = x_e.at[:n, 128].set(1.0)
    sc_add_cut = _make_sc_seg("add", 256, 9, _NR, kidxP.shape[0], jnp.float32)
    cut_pad = sc_add_cut(x_e, kidxP, kdstP, kbases40, knbs40).reshape(_NR, 256)

    hl_cut = _cut_tc(cut_pad, x_pad, w_cut, b_cut.reshape(1, 128))

    hl_cat = jnp.concatenate([hl_ego, hl_cut], axis=1)
    sc_add_mp = _make_sc_seg("add", 256, 16, _NR, srcP.shape[0], jnp.float32)
    agg_pad = sc_add_mp(hl_cat, srcP, dstmpP, bases40, nbs40).reshape(_NR, 256)

    out = _final_tc(agg_pad, x_pad, w_glob, b_glob.reshape(1, 128),
                    w_fc, b_fc.reshape(1, 128))
    return out[:n]
